# baseline (speedup 1.0000x reference)
import functools

import jax
import jax.numpy as jnp
from jax import lax
from jax.experimental import pallas as pl
from jax.experimental.pallas import tpu as pltpu

_K = 512
_D = 96
_ROWS = 8 * 96 * 110 * 110 // _D
_T = 2200
_STEPS = _ROWS // _T


def _vq_body(flat_ref, cbT_ref, cb_ref, c2_ref, a2_ref, q_ref, loss_ref,
             perp_ref, sse_ref, counts_ref):
    i = pl.program_id(0)
    a = flat_ref[:, :]
    prod = jnp.dot(a, cbT_ref[:, :], preferred_element_type=jnp.float32)
    a2 = a2_ref[:, :]
    d2 = jnp.sqrt(jnp.maximum(a2 - 2.0 * prod + c2_ref[:, :], 0.0))
    m = jnp.min(d2, axis=1, keepdims=True)
    col = lax.broadcasted_iota(jnp.int32, d2.shape, 1)
    idx = jnp.min(jnp.where(d2 == m, col, _K), axis=1, keepdims=True)
    oh = (col == idx).astype(jnp.float32)
    q = jnp.dot(oh, cb_ref[:, :], preferred_element_type=jnp.float32)
    q_ref[:, :] = q
    diff = q - a
    sse_t = jnp.reshape(jnp.sum(diff * diff), (1, 1))
    cnt_t = jnp.sum(oh, axis=0, keepdims=True)

    @pl.when(i == 0)
    def _init():
        sse_ref[:, :] = jnp.zeros_like(sse_ref)
        counts_ref[:, :] = jnp.zeros_like(counts_ref)

    sse_ref[:, :] += sse_t
    counts_ref[:, :] += cnt_t

    @pl.when(i == _STEPS - 1)
    def _fin():
        mse = sse_ref[0, 0] / jnp.float32(_ROWS * _D)
        loss_ref[:, :] = jnp.reshape(1.25 * mse, (1, 1))
        p = counts_ref[:, :] / jnp.float32(_ROWS)
        ent = jnp.sum(p * jnp.log(p + 1e-10))
        perp_ref[:, :] = jnp.reshape(jnp.exp(-ent), (1, 1))


@functools.partial(jax.jit)
def _vq(flat, codebook):
    cbT = codebook.T
    c2 = jnp.sum(codebook * codebook, axis=1)[None, :]
    a2 = jnp.sum(flat ** 2, axis=1, keepdims=True)
    q, loss, perp = pl.pallas_call(
        _vq_body,
        grid=(_STEPS,),
        in_specs=[
            pl.BlockSpec((_T, _D), lambda i: (i, 0)),
            pl.BlockSpec((_D, _K), lambda i: (0, 0)),
            pl.BlockSpec((_K, _D), lambda i: (0, 0)),
            pl.BlockSpec((1, _K), lambda i: (0, 0)),
            pl.BlockSpec((_T, 1), lambda i: (i, 0)),
        ],
        out_specs=[
            pl.BlockSpec((_T, _D), lambda i: (i, 0)),
            pl.BlockSpec((1, 1), lambda i: (0, 0)),
            pl.BlockSpec((1, 1), lambda i: (0, 0)),
        ],
        out_shape=[
            jax.ShapeDtypeStruct((_ROWS, _D), jnp.float32),
            jax.ShapeDtypeStruct((1, 1), jnp.float32),
            jax.ShapeDtypeStruct((1, 1), jnp.float32),
        ],
        scratch_shapes=[
            pltpu.VMEM((1, 1), jnp.float32),
            pltpu.VMEM((1, _K), jnp.float32),
        ],
        compiler_params=pltpu.CompilerParams(
            dimension_semantics=("arbitrary",),
        ),
    )(flat, cbT, codebook, c2, a2)
    return q, loss[0, 0], perp[0, 0]


def _conv(x, w, b, stride):
    y = lax.conv_general_dilated(x, w, (stride, stride), 'VALID',
                                 dimension_numbers=('NCHW', 'OIHW', 'NCHW'))
    return y + b[None, :, None, None]


def _deconv(x, w, b, stride):
    y = lax.conv_transpose(x, w, (stride, stride), 'VALID',
                           dimension_numbers=('NCHW', 'OIHW', 'NCHW'),
                           transpose_kernel=True)
    return y + b[None, :, None, None]


def kernel(x, conv1_w, conv1_b, conv2_w, conv2_b, codebook,
           deconv1_w, deconv1_b, deconv2_w, deconv2_b):
    z = jax.nn.relu(_conv(x, conv1_w, conv1_b, 2))
    z = jax.nn.relu(_conv(z, conv2_w, conv2_b, 1))
    flat = z.reshape(-1, _D)
    q, loss, perp = _vq(flat, codebook)
    quantized = q.reshape(z.shape)
    h = jax.nn.relu(_deconv(quantized, deconv1_w, deconv1_b, 1))
    x_recon = _deconv(h, deconv2_w, deconv2_b, 2)
    return (x_recon, loss, perp)
